# baseline (speedup 1.0000x reference)
import functools

import jax
import jax.numpy as jnp
from jax import lax
from jax.experimental import pallas as pl
from jax.experimental.pallas import tpu as pltpu
from jax.experimental.pallas import tpu_sc as plsc

H = 512
HALF = 384

_info = plsc.get_sparse_core_info()
_NC, _NS, _L = _info.num_cores, _info.num_subcores, _info.num_lanes
_NW = _NC * _NS


def _make_gather(n_rows: int):
  per_w = n_rows // _NW
  R = 80
  NB = 4
  G = per_w // R
  assert per_w % R == 0 and R % 8 == 0 and G >= 2 * NB
  mesh = plsc.VectorSubcoreMesh(core_axis_name="c", subcore_axis_name="s")

  @functools.partial(
      pl.kernel,
      mesh=mesh,
      out_type=jax.ShapeDtypeStruct((n_rows, HALF), jnp.float32),
      scratch_types=[
          pltpu.VMEM((per_w,), jnp.int32),
          pltpu.VMEM((NB, R, HALF), jnp.float32),
          pltpu.SemaphoreType.DMA,
          pltpu.SemaphoreType.DMA,
          pltpu.SemaphoreType.DMA,
          pltpu.SemaphoreType.DMA,
          pltpu.SemaphoreType.DMA,
          pltpu.SemaphoreType.DMA,
          pltpu.SemaphoreType.DMA,
          pltpu.SemaphoreType.DMA,
      ],
  )
  def k(table_hbm, idx_hbm, out_hbm, idx_v, rows_v,
        g0, g1, g2, g3, w0, w1, w2, w3):
    wid = lax.axis_index("s") * _NC + lax.axis_index("c")
    base = wid * per_w
    gsem = (g0, g1, g2, g3)
    wsem = (w0, w1, w2, w3)

    pltpu.sync_copy(idx_hbm.at[pl.ds(base, per_w)], idx_v)
    offs = (lax.iota(jnp.int32, _L) & 1) * H

    @pl.loop(0, per_w, step=_L)
    def _(i):
      sl = pl.ds(i, _L)
      idx_v[sl] = idx_v[sl] + offs

    def start_gather(g, b):
      return pltpu.async_copy(
          table_hbm.at[idx_v.at[pl.ds(g * R, R)]], rows_v.at[b], gsem[b])

    def wait_gather(b):
      pltpu.make_async_copy(
          table_hbm.at[idx_v.at[pl.ds(0, R)]], rows_v.at[b], gsem[b]).wait()

    def start_write(g, b):
      return pltpu.async_copy(
          rows_v.at[b], out_hbm.at[pl.ds(base + g * R, R)], wsem[b])

    def wait_write(b):
      pltpu.make_async_copy(
          rows_v.at[b], out_hbm.at[pl.ds(base, R)], wsem[b]).wait()

    start_gather(0, 0)
    for g in range(1, NB):
      start_gather(g, g)
      wait_gather(g - 1)
      start_write(g - 1, g - 1)

    def step(g, b):
      wait_write(b)
      start_gather(g, b)
      pb = (b + NB - 1) % NB
      wait_gather(pb)
      start_write(g - 1, pb)

    main = (G - NB) - (G - NB) % NB

    @pl.loop(NB, NB + main, step=NB)
    def _(o):
      for b in range(NB):
        step(o + b, b)

    for g in range(NB + main, G):
      step(g, g % NB)

    lb = (G - 1) % NB
    wait_gather(lb)
    start_write(G - 1, lb)
    for b in range(NB):
      wait_write(b)

  return k


_gather = _make_gather(2 * 1024 * 50)


def kernel(token, row_embed, col_embed):
  B, L, _ = token.shape
  table = jnp.concatenate([row_embed, col_embed], axis=0)
  idx = token.astype(jnp.int32).reshape(-1)
  out = _gather(table, idx)
  return out.reshape(B, L, 2 * HALF)


# speedup vs baseline: 1.0263x; 1.0263x over previous
import functools

import jax
import jax.numpy as jnp
from jax import lax
from jax.experimental import pallas as pl
from jax.experimental.pallas import tpu as pltpu
from jax.experimental.pallas import tpu_sc as plsc

H = 512
HALF = 384

_info = plsc.get_sparse_core_info()
_NC, _NS, _L = _info.num_cores, _info.num_subcores, _info.num_lanes
_NW = _NC * _NS


def _make_gather(n_rows: int):
  R = 64
  NB = 2
  G = n_rows // (_NW * R)
  assert n_rows == G * _NW * R and (G - NB) % NB == 0 and G >= 2 * NB
  mesh = plsc.VectorSubcoreMesh(core_axis_name="c", subcore_axis_name="s")

  @functools.partial(
      pl.kernel,
      mesh=mesh,
      out_type=jax.ShapeDtypeStruct((G, _NC, _NS, R, HALF), jnp.float32),
      scratch_types=[
          pltpu.VMEM((G, R), jnp.int32),
          pltpu.VMEM((NB, R, HALF), jnp.float32),
          pltpu.VMEM_SHARED((NB, _NS, R, HALF), jnp.float32),
          pltpu.SemaphoreType.DMA,
          pltpu.SemaphoreType.DMA,
          pltpu.SemaphoreType.DMA,
          pltpu.SemaphoreType.DMA,
      ],
  )
  def k(table_hbm, idx_hbm, out_hbm, idx_v, rows_v, shared, g0, g1, w0, w1):
    cid = lax.axis_index("c")
    sid = lax.axis_index("s")
    gsem = (g0, g1)
    wsem = (w0, w1)

    pltpu.sync_copy(idx_hbm.at[:, cid, sid], idx_v)
    offs = (lax.iota(jnp.int32, _L) & 1) * H

    @pl.loop(0, G)
    def _(g):
      @pl.loop(0, R, step=_L)
      def _(i):
        sl = pl.ds(i, _L)
        idx_v[g, sl] = idx_v[g, sl] + offs

    def start_gather(g, b):
      return pltpu.async_copy(
          table_hbm.at[idx_v.at[g]], rows_v.at[b], gsem[b])

    def wait_gather(b):
      pltpu.make_async_copy(
          table_hbm.at[idx_v.at[0]], rows_v.at[b], gsem[b]).wait()

    def start_write(g, b):
      return pltpu.async_copy(shared.at[b], out_hbm.at[g, cid], wsem[b])

    def wait_write(b):
      pltpu.make_async_copy(
          shared.at[b], out_hbm.at[0, cid], wsem[b]).wait()

    def round_body(g, b, drain, prefetch):
      wait_gather(b)
      if drain:
        @pl.when(sid == 0)
        def _():
          wait_write(b)
      plsc.subcore_barrier()
      pltpu.sync_copy(rows_v.at[b], shared.at[b, sid])
      if prefetch:
        start_gather(g + NB, b)
      plsc.subcore_barrier()

      @pl.when(sid == 0)
      def _():
        start_write(g, b)

    for b in range(NB):
      start_gather(b, b)
    for g in range(NB):
      round_body(g, g % NB, drain=False, prefetch=True)

    @pl.loop(NB, G - NB, step=NB)
    def _(o):
      for b in range(NB):
        round_body(o + b, b, drain=True, prefetch=True)

    for g in range(G - NB, G):
      round_body(g, g % NB, drain=True, prefetch=False)

    @pl.when(sid == 0)
    def _():
      for b in range(NB):
        wait_write(b)

    plsc.subcore_barrier()

  return k


_gather = _make_gather(2 * 1024 * 50)
_G = 2 * 1024 * 50 // (_NW * 64)


def kernel(token, row_embed, col_embed):
  B, L, _ = token.shape
  table = jnp.concatenate([row_embed, col_embed], axis=0)
  idx = token.astype(jnp.int32).reshape(_G, _NC, _NS, 64)
  out = _gather(table, idx)
  return out.reshape(B, L, 2 * HALF)


# speedup vs baseline: 1.1602x; 1.1305x over previous
import functools

import jax
import jax.numpy as jnp
from jax import lax
from jax.experimental import pallas as pl
from jax.experimental.pallas import tpu as pltpu
from jax.experimental.pallas import tpu_sc as plsc

H = 512
HALF = 384

_info = plsc.get_sparse_core_info()
_NC, _NS, _L = _info.num_cores, _info.num_subcores, _info.num_lanes
_NW = _NC * _NS


def _make_gather(n_rows: int):
  R = 64
  NB = 2
  G = n_rows // (_NW * R)
  assert n_rows == G * _NW * R and (G - NB) % NB == 0 and G >= 2 * NB
  mesh = plsc.VectorSubcoreMesh(core_axis_name="c", subcore_axis_name="s")

  @functools.partial(
      pl.kernel,
      mesh=mesh,
      out_type=jax.ShapeDtypeStruct((G, _NC, _NS, R, HALF), jnp.float32),
      scratch_types=[
          pltpu.VMEM((G, R), jnp.int32),
          pltpu.VMEM((NB, R, HALF), jnp.float32),
          pltpu.VMEM_SHARED((NB, _NS, R, HALF), jnp.float32),
          pltpu.SemaphoreType.DMA,
          pltpu.SemaphoreType.DMA,
          pltpu.SemaphoreType.DMA,
          pltpu.SemaphoreType.DMA,
      ],
  )
  def k(table_hbm, idx_hbm, out_hbm, idx_v, rows_v, shared, g0, g1, w0, w1):
    cid = lax.axis_index("c")
    sid = lax.axis_index("s")
    gsem = (g0, g1)
    wsem = (w0, w1)

    pltpu.sync_copy(idx_hbm.at[:, cid, sid], idx_v)
    offs = (lax.iota(jnp.int32, _L) & 1) * H

    @pl.loop(0, G)
    def _(g):
      @pl.loop(0, R, step=_L)
      def _(i):
        sl = pl.ds(i, _L)
        idx_v[g, sl] = idx_v[g, sl] + offs

    def start_gather(g, b):
      return pltpu.async_copy(
          table_hbm.at[idx_v.at[g]], rows_v.at[b], gsem[b])

    def wait_gather(b):
      pltpu.make_async_copy(
          table_hbm.at[idx_v.at[0]], rows_v.at[b], gsem[b]).wait()

    def start_write(g, b):
      return pltpu.async_copy(shared.at[b], out_hbm.at[g, cid], wsem[b])

    def wait_write(b):
      pltpu.make_async_copy(
          shared.at[b], out_hbm.at[0, cid], wsem[b]).wait()

    def round_body(g, b, drain, prefetch):
      wait_gather(b)
      if prefetch:
        start_gather(g + NB, b)
      if drain is None:
        @pl.when(sid == 0)
        def _():
          pltpu.sync_copy(rows_v.at[b], shared.at[b, sid])
          start_write(0, b).wait()

    for b in range(NB):
      start_gather(b, b)
    round_body(0, 0, drain=None, prefetch=True)
    round_body(1, 1, drain=False, prefetch=True)

    @pl.loop(NB, G - NB, step=NB)
    def _(o):
      for b in range(NB):
        round_body(o + b, b, drain=True, prefetch=True)

    for g in range(G - NB, G):
      round_body(g, g % NB, drain=True, prefetch=False)

    plsc.subcore_barrier()

  return k


_gather = _make_gather(2 * 1024 * 50)
_G = 2 * 1024 * 50 // (_NW * 64)


def kernel(token, row_embed, col_embed):
  B, L, _ = token.shape
  table = jnp.concatenate([row_embed, col_embed], axis=0)
  idx = token.astype(jnp.int32).reshape(_G, _NC, _NS, 64)
  out = _gather(table, idx)
  return out.reshape(B, L, 2 * HALF)


# speedup vs baseline: 1.1863x; 1.0224x over previous
import functools

import jax
import jax.numpy as jnp
from jax import lax
from jax.experimental import pallas as pl
from jax.experimental.pallas import tpu as pltpu
from jax.experimental.pallas import tpu_sc as plsc

FULL = 768
T = 160

_info = plsc.get_sparse_core_info()
_NC, _NS, _L = _info.num_cores, _info.num_subcores, _info.num_lanes


def _make_probe(n_tok: int):
  per_core = n_tok // _NC
  ROUNDS = per_core // T
  assert per_core % T == 0 and ROUNDS % 2 == 0
  mesh = plsc.VectorSubcoreMesh(core_axis_name="c", subcore_axis_name="s")

  @functools.partial(
      pl.kernel,
      mesh=mesh,
      out_type=jax.ShapeDtypeStruct((n_tok, FULL), jnp.float32),
      scratch_types=[
          pltpu.VMEM_SHARED((2, T, FULL), jnp.float32),
          pltpu.SemaphoreType.DMA,
          pltpu.SemaphoreType.DMA,
      ],
  )
  def k(table8_hbm, tok2_hbm, out_hbm, shared, w0, w1):
    cid = lax.axis_index("c")
    sid = lax.axis_index("s")
    base = cid * per_core
    wsem = (w0, w1)

    def start_write(r, b):
      return pltpu.async_copy(
          shared.at[b], out_hbm.at[pl.ds(base + r * T, T)], wsem[b])

    def wait_write(b):
      pltpu.make_async_copy(
          shared.at[b], out_hbm.at[pl.ds(base, T)], wsem[b]).wait()

    @pl.when(sid == 0)
    def _():
      start_write(0, 0)
      start_write(1, 1)

      @pl.loop(2, ROUNDS, step=2)
      def _(o):
        wait_write(0)
        start_write(o, 0)
        wait_write(1)
        start_write(o + 1, 1)

      wait_write(0)
      wait_write(1)

    plsc.subcore_barrier()

  return k


_probe = _make_probe(1024 * 50)


def kernel(token, row_embed, col_embed):
  B, L, _ = token.shape
  n_tok = B * L
  stacked = jnp.stack([row_embed, col_embed])
  table8 = stacked.reshape(2, 512, 4, 96).transpose(0, 2, 1, 3).reshape(
      8, 512, 96)
  tok2 = token.astype(jnp.int32).reshape(n_tok, 2).T
  out = _probe(table8, tok2)
  return out.reshape(B, L, FULL)


# speedup vs baseline: 1.2201x; 1.0285x over previous
import functools

import jax
import jax.numpy as jnp
from jax import lax
from jax.experimental import pallas as pl
from jax.experimental.pallas import tpu as pltpu

H = 512
HALF = 384
FULL = 2 * HALF


def _tc_lookup(t0, t1, row_embed, col_embed):
  N = t0.shape[0]
  BT = 512
  grid = N // BT

  def body(t0_ref, t1_ref, row_ref, col_ref, o_ref):
    iota = lax.broadcasted_iota(jnp.int32, (BT, H), 1)
    oh0 = (t0_ref[:][:, None] == iota).astype(jnp.float32)
    oh1 = (t1_ref[:][:, None] == iota).astype(jnp.float32)
    o_ref[:, :HALF] = jnp.dot(
        oh0, row_ref[:], preferred_element_type=jnp.float32)
    o_ref[:, HALF:] = jnp.dot(
        oh1, col_ref[:], preferred_element_type=jnp.float32)

  return pl.pallas_call(
      body,
      grid=(grid,),
      in_specs=[
          pl.BlockSpec((BT,), lambda i: (i,)),
          pl.BlockSpec((BT,), lambda i: (i,)),
          pl.BlockSpec((H, HALF), lambda i: (0, 0)),
          pl.BlockSpec((H, HALF), lambda i: (0, 0)),
      ],
      out_specs=pl.BlockSpec((BT, FULL), lambda i: (i, 0)),
      out_shape=jax.ShapeDtypeStruct((N, FULL), jnp.float32),
  )(t0, t1, row_embed, col_embed)


def kernel(token, row_embed, col_embed):
  B, L, _ = token.shape
  n_tok = B * L
  tok = token.astype(jnp.int32).reshape(n_tok, 2)
  out = _tc_lookup(tok[:, 0], tok[:, 1], row_embed, col_embed)
  return out.reshape(B, L, FULL)
